# baseline (speedup 1.0000x reference)
import jax
import jax.numpy as jnp
from jax.experimental import pallas as pl
from jax.experimental.pallas import tpu as pltpu

B = 2
Q = 900
C = 2
K = 100
NKP = 144
NVERT = 10475
NPOSE = 159
VD = NVERT * 3
QC = Q * C
SUB = 8
LAN = QC // SUB


def _select_body(logits_ref, boxes_ref, lh_ref, rh_ref, fc_ref, pose_ref,
                 beta_ref, expr_ref, cam_ref, kx_ref, ky_ref, kz_ref,
                 ts_ref, img_ref,
                 scores_ref, labels_ref, tkbox_ref, bsel_ref, lho_ref,
                 rho_ref, fco_ref, pose_o_ref, beta_o_ref, expr_o_ref,
                 px_ref, py_ref, gx_ref, gy_ref, gz_ref, transl_ref):
    p = jax.nn.sigmoid(logits_ref[0])
    flat = (jax.lax.broadcasted_iota(jnp.int32, (SUB, LAN), 0) * LAN
            + jax.lax.broadcasted_iota(jnp.int32, (SUB, LAN), 1))
    lane128 = jax.lax.broadcasted_iota(jnp.int32, (1, 128), 1)
    sub128 = jax.lax.broadcasted_iota(jnp.int32, (128, 1), 0)

    def body(k, carry):
        p, s_row, i_row, i_col = carry
        m = jnp.max(p)
        cand = jnp.where(p == m, flat, QC + 1)
        idx = jnp.min(cand)
        s_row = jnp.where(lane128 == k, m, s_row)
        i_row = jnp.where(lane128 == k, idx, i_row)
        i_col = jnp.where(sub128 == k, idx, i_col)
        p = jnp.where(flat == idx, -2.0, p)
        return p, s_row, i_row, i_col

    init = (p,
            jnp.zeros((1, 128), jnp.float32),
            jnp.full((1, 128), -1, jnp.int32),
            jnp.full((128, 1), -1, jnp.int32))
    _, s_row, i_row, i_col = jax.lax.fori_loop(0, K, body, init)

    scores_ref[0] = s_row[:, :K]
    labels_ref[0] = jnp.where(i_row[:, :K] >= 0, i_row[:, :K] % C, 0)
    tk_row = jnp.where(i_row[:, :K] >= 0, i_row[:, :K] // C, 0)
    tkbox_ref[0] = tk_row

    tk_col = i_col // C
    onehot = jnp.where(
        tk_col == jax.lax.broadcasted_iota(jnp.int32, (128, Q), 1),
        1.0, 0.0).astype(jnp.float32)

    def mm(ref):
        return jax.lax.dot_general(
            onehot, ref[0], (((1,), (0,)), ((), ())),
            precision=jax.lax.Precision.HIGHEST,
            preferred_element_type=jnp.float32)

    img_h = ts_ref[0, 0, 0]
    img_w = ts_ref[0, 0, 1]
    cc_x = img_ref[0, 0, 1] * 0.5
    cc_y = img_ref[0, 0, 0] * 0.5

    def conv_scale(g):
        cx = g[:, 0:1]
        cy = g[:, 1:2]
        w = g[:, 2:3]
        h = g[:, 3:4]
        return jnp.concatenate([
            (cx - 0.5 * w) * img_w,
            (cy - 0.5 * h) * img_h,
            (cx + 0.5 * w) * img_w,
            (cy + 0.5 * h) * img_h,
        ], axis=1)

    bsel_ref[0] = conv_scale(mm(boxes_ref))[:K]
    lho_ref[0] = conv_scale(mm(lh_ref))[:K]
    rho_ref[0] = conv_scale(mm(rh_ref))[:K]
    fco_ref[0] = conv_scale(mm(fc_ref))[:K]
    pose_o_ref[0] = mm(pose_ref)[:K]
    beta_o_ref[0] = mm(beta_ref)[:K]
    expr_o_ref[0] = mm(expr_ref)[:K]

    gcam = mm(cam_ref)
    s = gcam[:, 0:1] + 1e-9
    txs = gcam[:, 1:2] / s
    tys = gcam[:, 2:3] / s
    invs = 1.0 / s
    transl_ref[0] = jnp.concatenate([txs, tys, invs], axis=1)[:K]

    gx = mm(kx_ref)
    gy = mm(ky_ref)
    gz = mm(kz_ref)
    gx_ref[0] = gx[:K]
    gy_ref[0] = gy[:K]
    gz_ref[0] = gz[:K]
    zz = gz + invs
    px_ref[0] = ((gx + txs) / zz * 5000.0 + cc_x)[:K]
    py_ref[0] = ((gy + tys) / zz * 5000.0 + cc_y)[:K]


def _gather_body(idx_ref, src_ref, out_ref):
    out_ref[...] = src_ref[...]


def kernel(pred_logits, pred_boxes, pred_lhand_boxes, pred_rhand_boxes,
           pred_face_boxes, pred_smpl_fullpose, pred_smpl_beta,
           pred_smpl_expr, pred_smpl_cam, pred_smpl_kp3d, pred_smpl_verts,
           target_sizes, img_shape):
    logits3 = pred_logits.reshape(B, SUB, LAN)
    kx = pred_smpl_kp3d[..., 0]
    ky = pred_smpl_kp3d[..., 1]
    kz = pred_smpl_kp3d[..., 2]
    ts3 = target_sizes.reshape(B, 1, 2)
    img3 = img_shape.reshape(B, 1, 2)

    def bmap(b):
        return (b, 0, 0)

    in_specs = [
        pl.BlockSpec((1, SUB, LAN), bmap),
        pl.BlockSpec((1, Q, 4), bmap),
        pl.BlockSpec((1, Q, 4), bmap),
        pl.BlockSpec((1, Q, 4), bmap),
        pl.BlockSpec((1, Q, 4), bmap),
        pl.BlockSpec((1, Q, NPOSE), bmap),
        pl.BlockSpec((1, Q, 10), bmap),
        pl.BlockSpec((1, Q, 10), bmap),
        pl.BlockSpec((1, Q, 3), bmap),
        pl.BlockSpec((1, Q, NKP), bmap),
        pl.BlockSpec((1, Q, NKP), bmap),
        pl.BlockSpec((1, Q, NKP), bmap),
        pl.BlockSpec((1, 1, 2), bmap),
        pl.BlockSpec((1, 1, 2), bmap),
    ]
    out_shape = (
        jax.ShapeDtypeStruct((B, 1, K), jnp.float32),
        jax.ShapeDtypeStruct((B, 1, K), jnp.int32),
        jax.ShapeDtypeStruct((B, 1, K), jnp.int32),
        jax.ShapeDtypeStruct((B, K, 4), jnp.float32),
        jax.ShapeDtypeStruct((B, K, 4), jnp.float32),
        jax.ShapeDtypeStruct((B, K, 4), jnp.float32),
        jax.ShapeDtypeStruct((B, K, 4), jnp.float32),
        jax.ShapeDtypeStruct((B, K, NPOSE), jnp.float32),
        jax.ShapeDtypeStruct((B, K, 10), jnp.float32),
        jax.ShapeDtypeStruct((B, K, 10), jnp.float32),
        jax.ShapeDtypeStruct((B, K, NKP), jnp.float32),
        jax.ShapeDtypeStruct((B, K, NKP), jnp.float32),
        jax.ShapeDtypeStruct((B, K, NKP), jnp.float32),
        jax.ShapeDtypeStruct((B, K, NKP), jnp.float32),
        jax.ShapeDtypeStruct((B, K, NKP), jnp.float32),
        jax.ShapeDtypeStruct((B, K, 3), jnp.float32),
    )
    out_specs = [
        pl.BlockSpec((1, 1, K), bmap),
        pl.BlockSpec((1, 1, K), bmap),
        pl.BlockSpec((1, 1, K), bmap),
        pl.BlockSpec((1, K, 4), bmap),
        pl.BlockSpec((1, K, 4), bmap),
        pl.BlockSpec((1, K, 4), bmap),
        pl.BlockSpec((1, K, 4), bmap),
        pl.BlockSpec((1, K, NPOSE), bmap),
        pl.BlockSpec((1, K, 10), bmap),
        pl.BlockSpec((1, K, 10), bmap),
        pl.BlockSpec((1, K, NKP), bmap),
        pl.BlockSpec((1, K, NKP), bmap),
        pl.BlockSpec((1, K, NKP), bmap),
        pl.BlockSpec((1, K, NKP), bmap),
        pl.BlockSpec((1, K, NKP), bmap),
        pl.BlockSpec((1, K, 3), bmap),
    ]
    (scores3, labels3, tk3, bsel, lho, rho, fco, poseg, betag, exprg,
     px, py, gx, gy, gz, transl) = pl.pallas_call(
        _select_body,
        grid=(B,),
        in_specs=in_specs,
        out_specs=out_specs,
        out_shape=out_shape,
    )(logits3, pred_boxes, pred_lhand_boxes, pred_rhand_boxes,
      pred_face_boxes, pred_smpl_fullpose, pred_smpl_beta, pred_smpl_expr,
      pred_smpl_cam, kx, ky, kz, ts3, img3)

    scores = scores3.reshape(B, K)
    labels = labels3.reshape(B, K)
    tk = tk3.reshape(B, K)

    verts4 = pred_smpl_verts.reshape(B, Q, 1, VD)
    grid_spec = pltpu.PrefetchScalarGridSpec(
        num_scalar_prefetch=1,
        grid=(B, K),
        in_specs=[
            pl.BlockSpec((1, 1, 1, VD),
                         lambda b, i, idx: (b, idx[b, i], 0, 0)),
        ],
        out_specs=pl.BlockSpec((1, 1, 1, VD), lambda b, i, idx: (b, i, 0, 0)),
    )
    vsel = pl.pallas_call(
        _gather_body,
        grid_spec=grid_spec,
        out_shape=jax.ShapeDtypeStruct((B, K, 1, VD), jnp.float32),
    )(tk, verts4)
    smpl_verts = vsel.reshape(B, K, NVERT, 3)

    kp2d = jnp.stack([px, py], axis=-1)
    kp3d = jnp.stack([gx, gy, gz], axis=-1)
    root_pose = poseg[:, :, :3]
    body_pose = poseg[:, :, 3:66]
    lhand_pose = poseg[:, :, 66:111]
    rhand_pose = poseg[:, :, 111:156]
    jaw_pose = poseg[:, :, 156:]

    return (scores, labels, kp3d, root_pose, body_pose, lhand_pose,
            rhand_pose, jaw_pose, betag, exprg, kp2d, smpl_verts, transl,
            bsel, lho, rho, fco, bsel)
